# Initial kernel scaffold; baseline (speedup 1.0000x reference)
import jax
import jax.numpy as jnp
from jax.experimental import pallas as pl


def kernel(p, x, o, W, gamma, beta):
    raise NotImplementedError("write your pallas kernel here")


# speedup vs baseline: 8.7116x; 8.7116x over previous
import functools

import jax
import jax.numpy as jnp
from jax import lax
from jax.experimental import pallas as pl
from jax.experimental.pallas import tpu as pltpu
from jax.experimental.pallas import tpu_sc as plsc

LEAK = 0.2
EPS = 1e-5
K = 16


def _proj_body(x_ref, wc_ref, y_ref, z_ref, *, c_out):
    xb = x_ref[...]
    yz = lax.dot_general(xb, wc_ref[...], (((1,), (0,)), ((), ())),
                         preferred_element_type=jnp.float32,
                         precision=lax.Precision.HIGHEST)
    y_ref[...] = yz[:, :c_out]
    z_ref[...] = yz[:, c_out:]


def _knn_body(xb_ref, xs_ref, xst_ref, ys_ref, idx_ref, mx_ref, mn_ref,
              g_ref, q_ref, *, k, s_seg):
    seg = pl.program_id(0)
    xb = xb_ref[...]
    xs = xs_ref[...]
    xst = xst_ref[...]
    dots = lax.dot_general(xb, xs, (((1,), (1,)), ((), ())),
                           preferred_element_type=jnp.float32)
    sq_i = jnp.sum(xb * xb, axis=1, keepdims=True)
    sq_j = jnp.sum(xst * xst, axis=0, keepdims=True)
    d2 = (sq_i + sq_j) - 2.0 * dots
    iota = lax.broadcasted_iota(jnp.int32, d2.shape, 1)
    ys = ys_ref[...]
    big = jnp.float32(3.0e38)
    idx_cols = []
    mx = mn = g = q = None
    for _ in range(k):
        mv = jnp.min(d2, axis=1, keepdims=True)
        cand = jnp.where(d2 == mv, iota, s_seg)
        am = jnp.min(cand, axis=1, keepdims=True)
        onehot = iota == am
        d2 = jnp.where(onehot, big, d2)
        idx_cols.append(am)
        sel = lax.dot_general(onehot.astype(jnp.float32), ys,
                              (((1,), (0,)), ((), ())),
                              preferred_element_type=jnp.float32)
        if mx is None:
            mx, mn, g, q = sel, sel, sel, sel * sel
        else:
            mx = jnp.maximum(mx, sel)
            mn = jnp.minimum(mn, sel)
            g = g + sel
            q = q + sel * sel
    idx_ref[...] = jnp.concatenate(idx_cols, axis=1) + seg * s_seg
    mx_ref[...] = mx
    mn_ref[...] = mn
    g_ref[...] = g
    q_ref[...] = q


def _stats_body(g_ref, q_ref, z_ref, gam_ref, bet_ref, out_ref, acc_ref,
                *, nblocks, n_total, k):
    i = pl.program_id(0)

    @pl.when(i == 0)
    def _init():
        acc_ref[...] = jnp.zeros_like(acc_ref)

    g = g_ref[...]
    q = q_ref[...]
    z = z_ref[...]
    acc_ref[0:1, :] += jnp.sum(g, axis=0, keepdims=True)
    acc_ref[1:2, :] += jnp.sum(q, axis=0, keepdims=True)
    acc_ref[2:3, :] += jnp.sum(z * g, axis=0, keepdims=True)
    acc_ref[3:4, :] += jnp.sum(z, axis=0, keepdims=True)
    acc_ref[4:5, :] += jnp.sum(z * z, axis=0, keepdims=True)

    @pl.when(i == nblocks - 1)
    def _fin():
        cnt = jnp.float32(n_total * k)
        kf = jnp.float32(k)
        sum_g = acc_ref[0:1, :]
        sum_q = acc_ref[1:2, :]
        sum_zg = acc_ref[2:3, :]
        sum_z = acc_ref[3:4, :]
        sum_z2 = acc_ref[4:5, :]
        mean = (sum_g + kf * sum_z) / cnt
        e2 = (sum_q + 2.0 * sum_zg + kf * sum_z2) / cnt
        var = e2 - mean * mean
        scale = gam_ref[...] * lax.rsqrt(var + EPS)
        shift = bet_ref[...] - mean * scale
        out_ref[0:1, :] = scale
        out_ref[1:2, :] = shift


def _fin_body(z_ref, mx_ref, mn_ref, sc_ref, out_ref):
    scale = sc_ref[0:1, :]
    shift = sc_ref[1:2, :]
    m = jnp.where(scale >= 0.0, mx_ref[...], mn_ref[...])
    v = (z_ref[...] + m) * scale + shift
    out_ref[...] = jnp.where(v >= 0.0, v, LEAK * v)


def kernel(p, x, o, W, gamma, beta):
    n, d = x.shape
    bseg = o.shape[0]
    s_seg = n // bseg
    c_out = W.shape[1]
    k = K

    w1 = W[:d]
    w2 = W[d:]
    wc = jnp.concatenate([w1, w2 - w1], axis=1)
    xt = x.T

    rb = 512
    f32 = jnp.float32
    y, z = pl.pallas_call(
        functools.partial(_proj_body, c_out=c_out),
        grid=(n // rb,),
        in_specs=[
            pl.BlockSpec((rb, d), lambda i: (i, 0)),
            pl.BlockSpec((d, 2 * c_out), lambda i: (0, 0)),
        ],
        out_specs=[
            pl.BlockSpec((rb, c_out), lambda i: (i, 0)),
            pl.BlockSpec((rb, c_out), lambda i: (i, 0)),
        ],
        out_shape=[
            jax.ShapeDtypeStruct((n, c_out), f32),
            jax.ShapeDtypeStruct((n, c_out), f32),
        ],
    )(x, wc)

    r = 256
    nrb = s_seg // r
    idx, mx, mn, g, q = pl.pallas_call(
        functools.partial(_knn_body, k=k, s_seg=s_seg),
        grid=(bseg, nrb),
        in_specs=[
            pl.BlockSpec((r, d), lambda s, rr: (s * nrb + rr, 0)),
            pl.BlockSpec((s_seg, d), lambda s, rr: (s, 0)),
            pl.BlockSpec((d, s_seg), lambda s, rr: (0, s)),
            pl.BlockSpec((s_seg, c_out), lambda s, rr: (s, 0)),
        ],
        out_specs=[
            pl.BlockSpec((r, k), lambda s, rr: (s * nrb + rr, 0)),
            pl.BlockSpec((r, c_out), lambda s, rr: (s * nrb + rr, 0)),
            pl.BlockSpec((r, c_out), lambda s, rr: (s * nrb + rr, 0)),
            pl.BlockSpec((r, c_out), lambda s, rr: (s * nrb + rr, 0)),
            pl.BlockSpec((r, c_out), lambda s, rr: (s * nrb + rr, 0)),
        ],
        out_shape=[
            jax.ShapeDtypeStruct((n, k), jnp.int32),
            jax.ShapeDtypeStruct((n, c_out), f32),
            jax.ShapeDtypeStruct((n, c_out), f32),
            jax.ShapeDtypeStruct((n, c_out), f32),
            jax.ShapeDtypeStruct((n, c_out), f32),
        ],
    )(x, x, xt, y)

    nblocks = n // rb
    sc = pl.pallas_call(
        functools.partial(_stats_body, nblocks=nblocks, n_total=n, k=k),
        grid=(nblocks,),
        in_specs=[
            pl.BlockSpec((rb, c_out), lambda i: (i, 0)),
            pl.BlockSpec((rb, c_out), lambda i: (i, 0)),
            pl.BlockSpec((rb, c_out), lambda i: (i, 0)),
            pl.BlockSpec((1, c_out), lambda i: (0, 0)),
            pl.BlockSpec((1, c_out), lambda i: (0, 0)),
        ],
        out_specs=pl.BlockSpec((2, c_out), lambda i: (0, 0)),
        out_shape=jax.ShapeDtypeStruct((2, c_out), f32),
        scratch_shapes=[pltpu.VMEM((8, c_out), f32)],
    )(g, q, z, gamma.reshape(1, c_out), beta.reshape(1, c_out))

    out = pl.pallas_call(
        _fin_body,
        grid=(nblocks,),
        in_specs=[
            pl.BlockSpec((rb, c_out), lambda i: (i, 0)),
            pl.BlockSpec((rb, c_out), lambda i: (i, 0)),
            pl.BlockSpec((rb, c_out), lambda i: (i, 0)),
            pl.BlockSpec((2, c_out), lambda i: (0, 0)),
        ],
        out_specs=pl.BlockSpec((rb, c_out), lambda i: (i, 0)),
        out_shape=jax.ShapeDtypeStruct((n, c_out), f32),
    )(z, mx, mn, sc)
    return out
